# Initial kernel scaffold; baseline (speedup 1.0000x reference)
import functools

import jax
import jax.numpy as jnp
from jax import lax
from jax.experimental import pallas as pl
from jax.experimental.pallas import tpu as pltpu
from jax.experimental.pallas import tpu_sc as plsc

NC = 2
NS = 16
DEG_W = 16


def _sc_aggregate(h, src, dst, chunk):
    n, d = h.shape
    e = src.shape[0]
    epw = e // (NC * NS)
    iters = epw // chunk
    npt = n // NS
    assert epw * NC * NS == e and iters * chunk == epw and npt * NS == n
    assert chunk % 8 == 0 and epw % 8 == 0

    zero_rows = jnp.zeros((npt, d), jnp.float32)
    zero_deg = jnp.zeros((npt, DEG_W), jnp.float32)
    ones = jnp.ones((chunk, DEG_W), jnp.float32)

    @functools.partial(
        pl.kernel,
        out_type=[
            jax.ShapeDtypeStruct((NC, n, d), jnp.float32),
            jax.ShapeDtypeStruct((NC, n, DEG_W), jnp.float32),
        ],
        mesh=plsc.VectorSubcoreMesh(core_axis_name="c", subcore_axis_name="s"),
        scratch_types=[
            pltpu.VMEM((chunk,), jnp.int32),
            pltpu.VMEM((chunk,), jnp.int32),
            pltpu.VMEM((chunk, d), jnp.float32),
            pltpu.VMEM((chunk, DEG_W), jnp.float32),
            pltpu.VMEM_SHARED((n, d), jnp.float32),
            pltpu.VMEM_SHARED((n, DEG_W), jnp.float32),
            pltpu.SemaphoreType.DMA,
        ],
    )
    def k(h_hbm, src_hbm, dst_hbm, z_hbm, zd_hbm, ones_hbm,
          out_hbm, deg_hbm, sidx, didx, rows, onev, acc, dacc, sem):
        cid = lax.axis_index("c")
        sid = lax.axis_index("s")

        pltpu.sync_copy(z_hbm, acc.at[pl.ds(sid * npt, npt)])
        pltpu.sync_copy(zd_hbm, dacc.at[pl.ds(sid * npt, npt)])
        pltpu.sync_copy(ones_hbm, onev)
        plsc.subcore_barrier()

        base = (cid * NS + sid) * epw

        @pl.loop(0, iters)
        def _(i):
            off = pl.multiple_of(base + i * chunk, 8)
            pltpu.sync_copy(src_hbm.at[pl.ds(off, chunk)], sidx)
            pltpu.sync_copy(dst_hbm.at[pl.ds(off, chunk)], didx)
            pltpu.async_copy(h_hbm.at[sidx], rows, sem).wait()
            pltpu.sync_copy(rows, acc.at[didx], add=True)
            pltpu.sync_copy(onev, dacc.at[didx], add=True)

        plsc.subcore_barrier()
        sl = pl.ds(sid * npt, npt)
        pltpu.sync_copy(acc.at[sl], out_hbm.at[cid].at[sl])
        pltpu.sync_copy(dacc.at[sl], deg_hbm.at[cid].at[sl])

    return k(h, src, dst, zero_rows, zero_deg, ones)


def _fin_body(h_ref, p_ref, dg_ref, wt_ref, wrt_ref, b_ref, o_ref):
    s = p_ref[0] + p_ref[1]
    deg = dg_ref[0, :, 0:1] + dg_ref[1, :, 0:1]
    mean = s / jnp.maximum(deg, 1.0)
    agg = jnp.maximum(
        jnp.dot(mean, wt_ref[...], preferred_element_type=jnp.float32), 0.0)
    o_ref[...] = (agg
                  + jnp.dot(h_ref[...], wrt_ref[...],
                            preferred_element_type=jnp.float32)
                  + b_ref[...])


def _tc_finalize(h, p, dg, wt, wrt, b, block):
    n, d = h.shape
    grid = (n // block,)
    return pl.pallas_call(
        _fin_body,
        grid=grid,
        in_specs=[
            pl.BlockSpec((block, d), lambda i: (i, 0)),
            pl.BlockSpec((NC, block, d), lambda i: (0, i, 0)),
            pl.BlockSpec((NC, block, DEG_W), lambda i: (0, i, 0)),
            pl.BlockSpec((d, d), lambda i: (0, 0)),
            pl.BlockSpec((d, d), lambda i: (0, 0)),
            pl.BlockSpec((1, d), lambda i: (0, 0)),
        ],
        out_specs=pl.BlockSpec((block, d), lambda i: (i, 0)),
        out_shape=jax.ShapeDtypeStruct((n, d), jnp.float32),
    )(h, p, dg, wt, wrt, b)


def kernel(h, edge_index, W, W_res, b_res):
    ei = edge_index.astype(jnp.int32)
    src = ei[0]
    dst = ei[1]
    p, dg = _sc_aggregate(h, src, dst, chunk=400)
    return _tc_finalize(h, p, dg, W.T, W_res.T, b_res.reshape(1, -1),
                        block=1000)


# speedup vs baseline: 3.0967x; 3.0967x over previous
import functools

import jax
import jax.numpy as jnp
from jax import lax
from jax.experimental import pallas as pl
from jax.experimental.pallas import tpu as pltpu
from jax.experimental.pallas import tpu_sc as plsc

NS = 16
CHUNK = 128


def _sc_sum(h, src, dst, npad, npt, iters):
    d = h.shape[1]
    zero_rows = jnp.zeros((npt, d), jnp.float32)

    @functools.partial(
        pl.kernel,
        out_type=jax.ShapeDtypeStruct((npad, d), jnp.float32),
        mesh=plsc.VectorSubcoreMesh(core_axis_name="c", subcore_axis_name="s",
                                    num_cores=1),
        scratch_types=[
            pltpu.VMEM((CHUNK,), jnp.int32),
            pltpu.VMEM((CHUNK,), jnp.int32),
            pltpu.VMEM((CHUNK, d), jnp.float32),
            pltpu.VMEM_SHARED((npad, d), jnp.float32),
            pltpu.SemaphoreType.DMA,
        ],
    )
    def k(h_hbm, src_hbm, dst_hbm, z_hbm, out_hbm, sidx, didx, rows, acc, sem):
        sid = lax.axis_index("s")
        sl = pl.ds(sid * npt, npt)
        pltpu.sync_copy(z_hbm, acc.at[sl])
        plsc.subcore_barrier()

        @pl.loop(0, iters)
        def _(i):
            off = pl.multiple_of((i * NS + sid) * CHUNK, 128)
            pltpu.sync_copy(src_hbm.at[pl.ds(off, CHUNK)], sidx)
            pltpu.sync_copy(dst_hbm.at[pl.ds(off, CHUNK)], didx)
            pltpu.async_copy(h_hbm.at[sidx], rows, sem).wait()
            pltpu.sync_copy(rows, acc.at[didx], add=True)

        plsc.subcore_barrier()
        pltpu.sync_copy(acc.at[sl], out_hbm.at[sl])

    return k(h, src, dst, zero_rows)


def _sc_degree(dst, npad, npt, iters, d):
    zero_rows = jnp.zeros((npt, d), jnp.float32)
    ones = jnp.ones((CHUNK, d), jnp.float32)

    @functools.partial(
        pl.kernel,
        out_type=jax.ShapeDtypeStruct((npad, d), jnp.float32),
        mesh=plsc.VectorSubcoreMesh(core_axis_name="c", subcore_axis_name="s",
                                    num_cores=1),
        scratch_types=[
            pltpu.VMEM((CHUNK,), jnp.int32),
            pltpu.VMEM((CHUNK, d), jnp.float32),
            pltpu.VMEM_SHARED((npad, d), jnp.float32),
        ],
    )
    def k(dst_hbm, z_hbm, ones_hbm, out_hbm, didx, onev, acc):
        sid = lax.axis_index("s")
        sl = pl.ds(sid * npt, npt)
        pltpu.sync_copy(z_hbm, acc.at[sl])
        pltpu.sync_copy(ones_hbm, onev)
        plsc.subcore_barrier()

        @pl.loop(0, iters)
        def _(i):
            off = pl.multiple_of((i * NS + sid) * CHUNK, 128)
            pltpu.sync_copy(dst_hbm.at[pl.ds(off, CHUNK)], didx)
            pltpu.sync_copy(onev, acc.at[didx], add=True)

        plsc.subcore_barrier()
        pltpu.sync_copy(acc.at[sl], out_hbm.at[sl])

    return k(dst, zero_rows, ones)


def _fin_body(h_ref, p_ref, dg_ref, wt_ref, wrt_ref, b_ref, o_ref):
    inv = 1.0 / jnp.maximum(dg_ref[:, 0:1], 1.0)
    agg = jnp.maximum(
        jnp.dot(p_ref[...] * inv, wt_ref[...],
                preferred_element_type=jnp.float32),
        0.0)
    o_ref[...] = (agg
                  + jnp.dot(h_ref[...], wrt_ref[...],
                            preferred_element_type=jnp.float32)
                  + b_ref[...])


def _tc_finalize(h, p, dg, wt, wrt, b, block):
    n, d = h.shape
    return pl.pallas_call(
        _fin_body,
        grid=(n // block,),
        in_specs=[
            pl.BlockSpec((block, d), lambda i: (i, 0)),
            pl.BlockSpec((block, d), lambda i: (i, 0)),
            pl.BlockSpec((block, d), lambda i: (i, 0)),
            pl.BlockSpec((d, d), lambda i: (0, 0)),
            pl.BlockSpec((d, d), lambda i: (0, 0)),
            pl.BlockSpec((1, d), lambda i: (0, 0)),
        ],
        out_specs=pl.BlockSpec((block, d), lambda i: (i, 0)),
        out_shape=jax.ShapeDtypeStruct((n, d), jnp.float32),
    )(h, p, dg, wt, wrt, b)


def kernel(h, edge_index, W, W_res, b_res):
    n, d = h.shape
    ei = edge_index.astype(jnp.int32)
    src = ei[0]
    dst = ei[1]
    e = src.shape[0]
    epad = -(-e // (NS * CHUNK)) * NS * CHUNK
    if epad != e:
        src = jnp.concatenate([src, jnp.zeros((epad - e,), jnp.int32)])
        dst = jnp.concatenate([dst, jnp.full((epad - e,), n, jnp.int32)])
    iters = epad // (NS * CHUNK)
    npad = -(-n // (8 * NS)) * 8 * NS
    npt = npad // NS

    p = _sc_sum(h, src, dst, npad, npt, iters)
    dg = _sc_degree(dst, npad, npt, iters, d)
    return _tc_finalize(h, p, dg, W.T, W_res.T, b_res.reshape(1, -1),
                        block=1000)


# speedup vs baseline: 4.5945x; 1.4837x over previous
import dataclasses
import functools

import jax
import jax.numpy as jnp
from jax import lax
from jax.experimental import pallas as pl
from jax.experimental.pallas import tpu as pltpu
from jax.experimental.pallas import tpu_sc as plsc

NS = 16
CHUNK = 128


def _sc_aggregate(h, src, dst, npad, npt, iters):
    d = h.shape[1]
    zero_rows = jnp.zeros((npt, d), jnp.float32)
    zero_hist = jnp.zeros((npad,), jnp.float32)

    cp = pltpu.CompilerParams()
    if "needs_layout_passes" in pltpu.CompilerParams.__dataclass_fields__:
        cp = dataclasses.replace(cp, needs_layout_passes=False)

    @functools.partial(
        pl.kernel,
        out_type=[
            jax.ShapeDtypeStruct((npad, d), jnp.float32),
            jax.ShapeDtypeStruct((NS, 1, npad), jnp.float32),
        ],
        mesh=plsc.VectorSubcoreMesh(core_axis_name="c", subcore_axis_name="s",
                                    num_cores=1),
        compiler_params=cp,
        scratch_types=[
            pltpu.VMEM((CHUNK,), jnp.int32),
            pltpu.VMEM((CHUNK,), jnp.int32),
            pltpu.VMEM((CHUNK,), jnp.int32),
            pltpu.VMEM((CHUNK,), jnp.int32),
            pltpu.VMEM((CHUNK, d), jnp.float32),
            pltpu.VMEM((npad,), jnp.float32),
            pltpu.VMEM_SHARED((npad, d), jnp.float32),
            pltpu.SemaphoreType.DMA,
        ],
    )
    def k(h_hbm, src_hbm, dst_hbm, z_hbm, zh_hbm, out_hbm, deg_hbm,
          sidx0, sidx1, didx0, didx1, rows, hist, acc, sem):
        sid = lax.axis_index("s")
        sl = pl.ds(sid * npt, npt)
        pltpu.sync_copy(z_hbm, acc.at[sl])
        pltpu.sync_copy(zh_hbm, hist)
        plsc.subcore_barrier()

        sidx = (sidx0, sidx1)
        didx = (didx0, didx1)
        ones16 = jnp.ones((16,), jnp.float32)

        def stage(c, p):
            off = pl.multiple_of(c * CHUNK, 128)
            pltpu.sync_copy(src_hbm.at[pl.ds(off, CHUNK)], sidx[p])
            pltpu.sync_copy(dst_hbm.at[pl.ds(off, CHUNK)], didx[p])

        def step(i, p):
            g = pltpu.async_copy(h_hbm.at[sidx[p]], rows, sem)
            stage((i + 1) * NS + sid, 1 - p)
            for j in range(CHUNK // 16):
                v = didx[p][pl.ds(j * 16, 16)]
                plsc.addupdate_scatter(hist, [v], ones16)
            g.wait()
            pltpu.sync_copy(rows, acc.at[didx[p]], add=True)

        stage(sid, 0)

        @pl.loop(0, iters, step=2)
        def _(i):
            step(i, 0)
            step(i + 1, 1)

        plsc.subcore_barrier()
        pltpu.sync_copy(acc.at[sl], out_hbm.at[sl])
        pltpu.sync_copy(hist, deg_hbm.at[sid].at[0])

    return k(h, src, dst, zero_rows, zero_hist)


def _fin_body(h_ref, p_ref, dg_ref, wt_ref, wrt_ref, b_ref, o_ref):
    deg = jnp.sum(dg_ref[...], axis=1, keepdims=True)
    inv = 1.0 / jnp.maximum(deg, 1.0)
    agg = jnp.maximum(
        jnp.dot(p_ref[...] * inv, wt_ref[...],
                preferred_element_type=jnp.float32),
        0.0)
    o_ref[...] = (agg
                  + jnp.dot(h_ref[...], wrt_ref[...],
                            preferred_element_type=jnp.float32)
                  + b_ref[...])


def _tc_finalize(h, p, dgt, wt, wrt, b, block):
    n, d = h.shape
    return pl.pallas_call(
        _fin_body,
        grid=(n // block,),
        in_specs=[
            pl.BlockSpec((block, d), lambda i: (i, 0)),
            pl.BlockSpec((block, d), lambda i: (i, 0)),
            pl.BlockSpec((block, NS), lambda i: (i, 0)),
            pl.BlockSpec((d, d), lambda i: (0, 0)),
            pl.BlockSpec((d, d), lambda i: (0, 0)),
            pl.BlockSpec((1, d), lambda i: (0, 0)),
        ],
        out_specs=pl.BlockSpec((block, d), lambda i: (i, 0)),
        out_shape=jax.ShapeDtypeStruct((n, d), jnp.float32),
    )(h, p, dgt, wt, wrt, b)


def kernel(h, edge_index, W, W_res, b_res):
    n, d = h.shape
    ei = edge_index.astype(jnp.int32)
    src = ei[0]
    dst = ei[1]
    e = src.shape[0]
    group = NS * CHUNK
    iters = -(-e // group)
    iters += iters % 2
    epad = (iters + 1) * group
    src = jnp.concatenate([src, jnp.zeros((epad - e,), jnp.int32)])
    dst = jnp.concatenate([dst, jnp.full((epad - e,), n, jnp.int32)])
    npad = -(-n // (8 * NS)) * 8 * NS
    npt = npad // NS
    assert iters % 2 == 0

    p, dg = _sc_aggregate(h, src, dst, npad, npt, iters)
    dgt = dg.reshape(NS, npad).T
    return _tc_finalize(h, p, dgt, W.T, W_res.T, b_res.reshape(1, -1),
                        block=1000)


# speedup vs baseline: 4.5999x; 1.0012x over previous
import dataclasses
import functools

import jax
import jax.numpy as jnp
from jax import lax
from jax.experimental import pallas as pl
from jax.experimental.pallas import tpu as pltpu
from jax.experimental.pallas import tpu_sc as plsc

NS = 16
CHUNK = 128


def _sc_aggregate(h, src, dst, npad, npt, iters):
    d = h.shape[1]
    zero_rows = jnp.zeros((npt, d), jnp.float32)
    zero_hist = jnp.zeros((npad,), jnp.float32)

    cp = pltpu.CompilerParams()
    if "needs_layout_passes" in pltpu.CompilerParams.__dataclass_fields__:
        cp = dataclasses.replace(cp, needs_layout_passes=False)

    @functools.partial(
        pl.kernel,
        out_type=[
            jax.ShapeDtypeStruct((npad, d), jnp.float32),
            jax.ShapeDtypeStruct((NS, 1, npad), jnp.float32),
        ],
        mesh=plsc.VectorSubcoreMesh(core_axis_name="c", subcore_axis_name="s",
                                    num_cores=1),
        compiler_params=cp,
        scratch_types=[
            pltpu.VMEM((CHUNK,), jnp.int32),
            pltpu.VMEM((CHUNK,), jnp.int32),
            pltpu.VMEM((CHUNK,), jnp.int32),
            pltpu.VMEM((CHUNK,), jnp.int32),
            pltpu.VMEM((CHUNK, d), jnp.float32),
            pltpu.VMEM((npad,), jnp.float32),
            pltpu.VMEM_SHARED((npad, d), jnp.float32),
            pltpu.SemaphoreType.DMA,
        ],
    )
    def k(h_hbm, src_hbm, dst_hbm, z_hbm, zh_hbm, out_hbm, deg_hbm,
          sidx0, sidx1, didx0, didx1, rows, hist, acc, sem):
        sid = lax.axis_index("s")
        sl = pl.ds(sid * npt, npt)
        pltpu.sync_copy(z_hbm, acc.at[sl])
        pltpu.sync_copy(zh_hbm, hist)
        plsc.subcore_barrier()

        sidx = (sidx0, sidx1)
        didx = (didx0, didx1)
        ones16 = jnp.ones((16,), jnp.float32)

        def stage(c, p):
            off = pl.multiple_of(c * CHUNK, 128)
            pltpu.sync_copy(src_hbm.at[pl.ds(off, CHUNK)], sidx[p])
            pltpu.sync_copy(dst_hbm.at[pl.ds(off, CHUNK)], didx[p])

        def step(i, p):
            g = pltpu.async_copy(h_hbm.at[sidx[p]], rows, sem)
            stage((i + 1) * NS + sid, 1 - p)
            for j in range(CHUNK // 16):
                v = didx[p][pl.ds(j * 16, 16)]
                plsc.addupdate_scatter(hist, [v], ones16)
            g.wait()
            pltpu.sync_copy(rows, acc.at[didx[p]], add=True)

        stage(sid, 0)

        @pl.loop(0, iters, step=2)
        def _(i):
            step(i, 0)
            step(i + 1, 1)

        plsc.subcore_barrier()
        pltpu.sync_copy(acc.at[sl], out_hbm.at[sl])
        pltpu.sync_copy(hist, deg_hbm.at[sid].at[0])

    return k(h, src, dst, zero_rows, zero_hist)


def _res_body(h_ref, wrt_ref, b_ref, o_ref):
    o_ref[...] = jnp.dot(h_ref[...], wrt_ref[...],
                         preferred_element_type=jnp.float32) + b_ref[...]


def _tc_residual(h, wrt, b, block):
    n, d = h.shape
    return pl.pallas_call(
        _res_body,
        grid=(n // block,),
        in_specs=[
            pl.BlockSpec((block, d), lambda i: (i, 0)),
            pl.BlockSpec((d, d), lambda i: (0, 0)),
            pl.BlockSpec((1, d), lambda i: (0, 0)),
        ],
        out_specs=pl.BlockSpec((block, d), lambda i: (i, 0)),
        out_shape=jax.ShapeDtypeStruct((n, d), jnp.float32),
    )(h, wrt, b)


def _fin_body(res_ref, p_ref, dg_ref, wt_ref, o_ref):
    deg = jnp.sum(dg_ref[...], axis=1, keepdims=True)
    inv = 1.0 / jnp.maximum(deg, 1.0)
    agg = jnp.maximum(
        jnp.dot(p_ref[...] * inv, wt_ref[...],
                preferred_element_type=jnp.float32),
        0.0)
    o_ref[...] = agg + res_ref[...]


def _tc_finalize(res, p, dgt, wt, block):
    n, d = res.shape
    return pl.pallas_call(
        _fin_body,
        grid=(n // block,),
        in_specs=[
            pl.BlockSpec((block, d), lambda i: (i, 0)),
            pl.BlockSpec((block, d), lambda i: (i, 0)),
            pl.BlockSpec((block, NS), lambda i: (i, 0)),
            pl.BlockSpec((d, d), lambda i: (0, 0)),
        ],
        out_specs=pl.BlockSpec((block, d), lambda i: (i, 0)),
        out_shape=jax.ShapeDtypeStruct((n, d), jnp.float32),
    )(res, p, dgt, wt)


def kernel(h, edge_index, W, W_res, b_res):
    n, d = h.shape
    ei = edge_index.astype(jnp.int32)
    src = ei[0]
    dst = ei[1]
    e = src.shape[0]
    group = NS * CHUNK
    iters = -(-e // group)
    iters += iters % 2
    epad = (iters + 1) * group
    src = jnp.concatenate([src, jnp.zeros((epad - e,), jnp.int32)])
    dst = jnp.concatenate([dst, jnp.full((epad - e,), n, jnp.int32)])
    npad = -(-n // (8 * NS)) * 8 * NS
    npt = npad // NS
    assert iters % 2 == 0

    res = _tc_residual(h, W_res.T, b_res.reshape(1, -1), block=1000)
    p, dg = _sc_aggregate(h, src, dst, npad, npt, iters)
    dgt = dg.reshape(NS, npad).T
    return _tc_finalize(res, p, dgt, W.T, block=1000)
